# baseline (speedup 1.0000x reference)
import functools

import jax
import jax.numpy as jnp
from jax import lax
from jax.experimental import pallas as pl
from jax.experimental.pallas import tpu as pltpu
from jax.experimental.pallas import tpu_sc as plsc

_NC = 2
_NS = 16
_BLOCK_B = 128
_SPLIT = 2048


def _tc_permute_kernel(perm_ref, x_ref, o_ref):
    xb = x_ref[...]
    idx = jnp.broadcast_to(perm_ref[0, :][None, None, :], xb.shape)
    o_ref[...] = jnp.take_along_axis(xb, idx, axis=2)


def _reverse_rows(xbuf, obuf, n_rows):
    def row_body(r, carry):
        for c in range(8):
            v = xbuf[r, pl.ds((7 - c) * 16, 16)]
            obuf[r, pl.ds(c * 16, 16)] = jnp.flip(v, axis=0)
        return carry

    lax.fori_loop(0, n_rows, row_body, 0, unroll=4)


def _sc_body(x_hbm, out_hbm, xb0, xb1, ob0, ob1, si0, si1, so0, so1):
    n_sc = out_hbm.shape[0]
    split = x_hbm.shape[0] - n_sc
    per_w = n_sc // (_NC * _NS)
    wid = lax.axis_index("s") * _NC + lax.axis_index("c")
    base = wid * per_w
    n_rows = x_hbm.shape[1]
    xbufs = (xb0, xb1)
    obufs = (ob0, ob1)
    sins = (si0, si1)
    souts = (so0, so1)

    def start_in(b, k):
        pltpu.async_copy(x_hbm.at[split + base + b], xbufs[k], sins[k])

    def wait_in(b, k):
        pltpu.make_async_copy(x_hbm.at[split + base + b], xbufs[k], sins[k]).wait()

    def start_out(b, k):
        pltpu.async_copy(obufs[k], out_hbm.at[base + b], souts[k])

    def wait_out(b, k):
        pltpu.make_async_copy(obufs[k], out_hbm.at[base + b], souts[k]).wait()

    start_in(0, 0)
    start_in(1, 1)
    wait_in(0, 0)
    _reverse_rows(xb0, ob0, n_rows)
    start_out(0, 0)
    start_in(2, 0)
    wait_in(1, 1)
    _reverse_rows(xb1, ob1, n_rows)
    start_out(1, 1)
    start_in(3, 1)

    def pair_body(j, carry):
        b = 2 + 2 * j
        for k in range(2):
            bb = b + k
            wait_in(bb, k)
            wait_out(bb - 2, k)
            _reverse_rows(xbufs[k], obufs[k], n_rows)
            start_out(bb, k)
            start_in(bb + 2, k)
        return carry

    lax.fori_loop(0, (per_w - 4) // 2, pair_body, 0)

    for k in range(2):
        bb = per_w - 2 + k
        wait_in(bb, k)
        wait_out(bb - 2, k)
        _reverse_rows(xbufs[k], obufs[k], n_rows)
        start_out(bb, k)
    for k in range(2):
        wait_out(per_w - 2 + k, k)


def kernel(x, perm):
    b, s, f = x.shape
    perm2 = perm.reshape(1, f)
    tc_half = pl.pallas_call(
        _tc_permute_kernel,
        grid=(_SPLIT // _BLOCK_B,),
        in_specs=[
            pl.BlockSpec((1, f), lambda i: (0, 0)),
            pl.BlockSpec((_BLOCK_B, s, f), lambda i: (i, 0, 0)),
        ],
        out_specs=pl.BlockSpec((_BLOCK_B, s, f), lambda i: (i, 0, 0)),
        out_shape=jax.ShapeDtypeStruct((_SPLIT, s, f), x.dtype),
    )(perm2, x)

    mesh = plsc.VectorSubcoreMesh(core_axis_name="c", subcore_axis_name="s")
    sc_fn = functools.partial(
        pl.kernel,
        mesh=mesh,
        out_type=jax.ShapeDtypeStruct((b - _SPLIT, s, f), x.dtype),
        scratch_types=[
            pltpu.VMEM((s, f), jnp.float32),
            pltpu.VMEM((s, f), jnp.float32),
            pltpu.VMEM((s, f), jnp.float32),
            pltpu.VMEM((s, f), jnp.float32),
            pltpu.SemaphoreType.DMA,
            pltpu.SemaphoreType.DMA,
            pltpu.SemaphoreType.DMA,
            pltpu.SemaphoreType.DMA,
        ],
    )(_sc_body)
    sc_half = sc_fn(x)
    return tc_half, sc_half


# speedup vs baseline: 1.1198x; 1.1198x over previous
import jax
import jax.numpy as jnp
from jax.experimental import pallas as pl


_BLOCK_B = 256


def _permute_kernel(perm_ref, x_ref, o_ref):
    xb = x_ref[...]
    idx = jnp.broadcast_to(perm_ref[0, :][None, None, :], xb.shape)
    o_ref[...] = jnp.take_along_axis(xb, idx, axis=2)


def kernel(x, perm):
    b, s, f = x.shape
    grid = (b // _BLOCK_B,)
    perm2 = perm.reshape(1, f)
    return pl.pallas_call(
        _permute_kernel,
        grid=grid,
        in_specs=[
            pl.BlockSpec((1, f), lambda i: (0, 0)),
            pl.BlockSpec((_BLOCK_B, s, f), lambda i: (i, 0, 0)),
        ],
        out_specs=pl.BlockSpec((_BLOCK_B, s, f), lambda i: (i, 0, 0)),
        out_shape=jax.ShapeDtypeStruct((b, s, f), x.dtype),
    )(perm2, x)
